# baseline (speedup 1.0000x reference)
import jax
import jax.numpy as jnp
from jax import lax
from jax.experimental import pallas as pl
from jax.experimental.pallas import tpu as pltpu
from jax.experimental.pallas import tpu_sc as plsc

_NC, _NS, _L = 2, 16, 16
_NW = _NC * _NS


def _body(t_hbm, beta_hbm, alpha_hbm, out_b_hbm, out_a_hbm,
          idx_v, ob_v, oa_v, sem_g, sem_out):
    wid = lax.axis_index("s") * _NC + lax.axis_index("c")
    bw = idx_v.shape[0]
    base = wid * bw
    pltpu.sync_copy(t_hbm.at[pl.ds(base, bw)], idx_v)
    g_b = pltpu.async_copy(beta_hbm.at[idx_v], ob_v, sem_g)
    g_a = pltpu.async_copy(alpha_hbm.at[idx_v], oa_v, sem_g)
    g_b.wait()
    co_b = pltpu.async_copy(ob_v, out_b_hbm.at[pl.ds(base, bw)], sem_out)
    g_a.wait()
    co_a = pltpu.async_copy(oa_v, out_a_hbm.at[pl.ds(base, bw)], sem_out)
    co_b.wait()
    co_a.wait()


def kernel(t, beta, alpha):
    b = t.shape[0]
    bw = b // _NW
    run = pl.kernel(
        _body,
        out_type=(jax.ShapeDtypeStruct((b,), jnp.float32),
                  jax.ShapeDtypeStruct((b,), jnp.float32)),
        mesh=plsc.VectorSubcoreMesh(core_axis_name="c", subcore_axis_name="s"),
        scratch_types=[
            pltpu.VMEM((bw,), jnp.int32),
            pltpu.VMEM((bw,), jnp.float32),
            pltpu.VMEM((bw,), jnp.float32),
            pltpu.SemaphoreType.DMA,
            pltpu.SemaphoreType.DMA,
        ],
        compiler_params=pltpu.CompilerParams(needs_layout_passes=False),
    )
    return run(t, beta, alpha)


# speedup vs baseline: 1.6581x; 1.6581x over previous
import jax
import jax.numpy as jnp
from jax import lax
from jax.experimental import pallas as pl
from jax.experimental.pallas import tpu as pltpu
from jax.experimental.pallas import tpu_sc as plsc

_NC, _NS, _L = 2, 16, 16
_TBL = 1024


def _body(t_hbm, tbl_hbm, out_hbm,
          idx_v, tbl_v, o_v, sem_in, sem_out):
    c = lax.axis_index("c")
    s = lax.axis_index("s")
    bw = idx_v.shape[0]
    base = s * bw
    cp_t = pltpu.async_copy(t_hbm.at[pl.ds(base, bw)], idx_v, sem_in)
    cp_tb = pltpu.async_copy(tbl_hbm.at[pl.ds(c * _TBL, _TBL)], tbl_v, sem_in)
    cp_t.wait()
    cp_tb.wait()
    for i in range(bw // _L):
        off = i * _L
        o_v[pl.ds(off, _L)] = plsc.load_gather(tbl_v, [idx_v[pl.ds(off, _L)]])
    nb = t_hbm.shape[0]
    pltpu.async_copy(o_v, out_hbm.at[pl.ds(c * nb + base, bw)], sem_out).wait()


def kernel(t, beta, alpha):
    b = t.shape[0]
    bw = b // _NS
    n = beta.shape[0]
    tbl = jnp.concatenate([jnp.pad(beta, (0, _TBL - n)),
                           jnp.pad(alpha, (0, _TBL - n))])
    run = pl.kernel(
        _body,
        out_type=jax.ShapeDtypeStruct((2 * b,), jnp.float32),
        mesh=plsc.VectorSubcoreMesh(core_axis_name="c", subcore_axis_name="s"),
        scratch_types=[
            pltpu.VMEM((bw,), jnp.int32),
            pltpu.VMEM((_TBL,), jnp.float32),
            pltpu.VMEM((bw,), jnp.float32),
            pltpu.SemaphoreType.DMA,
            pltpu.SemaphoreType.DMA,
        ],
        compiler_params=pltpu.CompilerParams(needs_layout_passes=False),
    )
    out = run(t, tbl)
    return out[:b], out[b:]


# speedup vs baseline: 1.7568x; 1.0595x over previous
import jax
import jax.numpy as jnp
from jax import lax
from jax.experimental import pallas as pl
from jax.experimental.pallas import tpu as pltpu
from jax.experimental.pallas import tpu_sc as plsc

_NC, _NS, _L = 2, 16, 16
_NW = _NC * _NS


def _body(t_hbm, beta_hbm, alpha_hbm, out_b_hbm, out_a_hbm,
          idx_v, beta_v, alpha_v, ob_v, oa_v, sem_in, sem_out):
    wid = lax.axis_index("s") * _NC + lax.axis_index("c")
    bw = idx_v.shape[0]
    base = wid * bw
    n = beta_hbm.shape[0]
    cp_t = pltpu.async_copy(t_hbm.at[pl.ds(base, bw)], idx_v, sem_in)
    cp_b = pltpu.async_copy(beta_hbm, beta_v.at[pl.ds(0, n)], sem_in)
    cp_a = pltpu.async_copy(alpha_hbm, alpha_v.at[pl.ds(0, n)], sem_in)
    cp_t.wait()
    cp_b.wait()
    cp_a.wait()

    def step(i, carry):
        off = i * _L
        idx = idx_v[pl.ds(off, _L)]
        ob_v[pl.ds(off, _L)] = plsc.load_gather(beta_v, [idx])
        oa_v[pl.ds(off, _L)] = plsc.load_gather(alpha_v, [idx])
        return carry

    lax.fori_loop(0, bw // _L, step, 0)
    co_b = pltpu.async_copy(ob_v, out_b_hbm.at[pl.ds(base, bw)], sem_out)
    co_a = pltpu.async_copy(oa_v, out_a_hbm.at[pl.ds(base, bw)], sem_out)
    co_b.wait()
    co_a.wait()


def kernel(t, beta, alpha):
    b = t.shape[0]
    bw = b // _NW
    tbl_pad = (beta.shape[0] + _L - 1) // _L * _L
    run = pl.kernel(
        _body,
        out_type=(jax.ShapeDtypeStruct((b,), jnp.float32),
                  jax.ShapeDtypeStruct((b,), jnp.float32)),
        mesh=plsc.VectorSubcoreMesh(core_axis_name="c", subcore_axis_name="s"),
        scratch_types=[
            pltpu.VMEM((bw,), jnp.int32),
            pltpu.VMEM((tbl_pad,), jnp.float32),
            pltpu.VMEM((tbl_pad,), jnp.float32),
            pltpu.VMEM((bw,), jnp.float32),
            pltpu.VMEM((bw,), jnp.float32),
            pltpu.SemaphoreType.DMA,
            pltpu.SemaphoreType.DMA,
        ],
        compiler_params=pltpu.CompilerParams(needs_layout_passes=False),
    )
    return run(t, beta, alpha)
